# baseline (speedup 1.0000x reference)
import functools

import jax
import jax.numpy as jnp
from jax import lax
from jax.experimental import pallas as pl
from jax.experimental.pallas import tpu as pltpu
from jax.experimental.pallas import tpu_sc as plsc

NTIME = 512
NCOMP = 16
MAX_EVENTS = 1024
D = 512
L = 4
H = 8
DH = D // H
DFF = 2048
TABW = 32


def _sc_event_build(xf, ztab, zmask):
    B = xf.shape[0]
    mesh = plsc.VectorSubcoreMesh(core_axis_name="c", subcore_axis_name="s")

    @functools.partial(
        pl.kernel,
        mesh=mesh,
        compiler_params=pltpu.CompilerParams(needs_layout_passes=False),
        out_type=(
            jax.ShapeDtypeStruct((B, MAX_EVENTS * TABW), jnp.float32),
            jax.ShapeDtypeStruct((B, MAX_EVENTS), jnp.float32),
        ),
        scratch_types=[
            pltpu.VMEM((NTIME * NCOMP,), jnp.float32),
            pltpu.VMEM((MAX_EVENTS * TABW,), jnp.float32),
            pltpu.VMEM((MAX_EVENTS,), jnp.float32),
        ],
    )
    def k(x_hbm, ztab_hbm, zmask_hbm, tab_hbm, mask_hbm, x_v, tab_v, msk_v):
        wid = lax.axis_index("s") * 2 + lax.axis_index("c")

        @pl.when(wid < B)
        def _():
            pltpu.sync_copy(x_hbm.at[wid], x_v)
            pltpu.sync_copy(ztab_hbm, tab_v)
            pltpu.sync_copy(zmask_hbm, msk_v)

            lanes = lax.iota(jnp.int32, 16)
            ones16 = jnp.ones((16,), jnp.float32)

            def body(t, carry):
                base, prev, last_start, last_rank = carry
                xv = x_v[pl.ds(t * 16, 16)]
                bits = (xv > 0.5).astype(jnp.int32)
                st = bits != prev
                sti = st.astype(jnp.int32)
                inc = plsc.cumsum(sti)
                rank = base + inc - sti
                valid = jnp.logical_and(st, rank < MAX_EVENTS)
                pm = jnp.logical_and(
                    st,
                    jnp.logical_and(last_rank >= 0, last_rank < MAX_EVENTS))
                durv = (t - last_start).astype(jnp.float32) * (1.0 / NTIME)
                plsc.store_scatter(tab_v, [last_rank * TABW + 18], durv,
                                   mask=pm)
                fl = rank * TABW
                plsc.store_scatter(tab_v, [fl + lanes], ones16, mask=valid)
                plsc.store_scatter(tab_v, [fl + 16],
                                   bits.astype(jnp.float32), mask=valid)
                tn = jnp.full((16,), 0.0, jnp.float32) + (
                    t.astype(jnp.float32) * (1.0 / (NTIME - 1)))
                plsc.store_scatter(tab_v, [fl + 17], tn, mask=valid)
                plsc.store_scatter(msk_v, [rank], ones16, mask=valid)
                nbase = base + jnp.sum(sti)
                nstart = jnp.where(st, t, last_start)
                nrank = jnp.where(st, rank, last_rank)
                return (nbase, bits, nstart, nrank)

            init = (jnp.int32(0),
                    jnp.full((16,), -1, jnp.int32),
                    jnp.zeros((16,), jnp.int32),
                    jnp.full((16,), -1, jnp.int32))
            base, prev, last_start, last_rank = lax.fori_loop(
                0, NTIME, body, init)
            pm = jnp.logical_and(last_rank >= 0, last_rank < MAX_EVENTS)
            durv = (NTIME - last_start).astype(jnp.float32) * (1.0 / NTIME)
            plsc.store_scatter(tab_v, [last_rank * TABW + 18], durv, mask=pm)

            pltpu.sync_copy(tab_v, tab_hbm.at[wid])
            pltpu.sync_copy(msk_v, mask_hbm.at[wid])

    return k(xf, ztab, zmask)


def _ln_tc(x, s, b):
    mu = jnp.mean(x, axis=-1, keepdims=True)
    var = jnp.mean((x - mu) ** 2, axis=-1, keepdims=True)
    return (x - mu) * lax.rsqrt(var + 1e-5) * s[None, :] + b[None, :]


def _attn_ffn_core(h, mc, wq_ref, wk_ref, wv_ref, wo_ref,
                   l1s_ref, l1b_ref, l2s_ref, l2b_ref,
                   w1_ref, b1_ref, w2_ref, b2_ref):
    bf = jnp.bfloat16
    hn = _ln_tc(h, l1s_ref[...], l1b_ref[...]).astype(bf)
    q = jnp.dot(hn, wq_ref[...], preferred_element_type=jnp.float32)
    k = jnp.dot(hn, wk_ref[...], preferred_element_type=jnp.float32)
    v = jnp.dot(hn, wv_ref[...], preferred_element_type=jnp.float32)
    vm = v * mc
    scale = 1.0 / (DH ** 0.5)
    qb = (q * scale).astype(bf)
    kb = k.astype(bf)
    outs = []
    for hh in range(H):
        sl = slice(hh * DH, (hh + 1) * DH)
        s = lax.dot_general(qb[:, sl], kb[:, sl],
                            (((1,), (1,)), ((), ())),
                            preferred_element_type=jnp.float32)
        eb = jnp.exp(s).astype(bf)
        va = jnp.concatenate([vm[:, sl], mc], axis=1).astype(bf)
        oa = jnp.dot(eb, va, preferred_element_type=jnp.float32)
        outs.append(oa[:, :DH] * (1.0 / oa[:, DH:DH + 1]))
    o = jnp.concatenate(outs, axis=1).astype(bf)
    h2 = h + jnp.dot(o, wo_ref[...], preferred_element_type=jnp.float32)
    hn2 = _ln_tc(h2, l2s_ref[...], l2b_ref[...]).astype(bf)
    t1 = jnp.dot(hn2, w1_ref[...], preferred_element_type=jnp.float32)
    t1 = jnp.maximum(t1 + b1_ref[...][None, :], 0.0).astype(bf)
    t2 = jnp.dot(t1, w2_ref[...], preferred_element_type=jnp.float32)
    return h2 + t2 + b2_ref[...][None, :]


def _first_body(tab_ref, mc_ref, wp_ref, bp_ref, *refs):
    out_ref = refs[-1]
    tab = tab_ref[0]
    mc = mc_ref[0]
    h = jnp.dot(tab, wp_ref[...], preferred_element_type=jnp.float32)
    h = h + bp_ref[...][None, :]
    out_ref[0] = _attn_ffn_core(h, mc, *refs[:-1]).astype(jnp.bfloat16)


def _mid_body(h_ref, mc_ref, *refs):
    out_ref = refs[-1]
    h = h_ref[0].astype(jnp.float32)
    mc = mc_ref[0]
    out_ref[0] = _attn_ffn_core(h, mc, *refs[:-1]).astype(jnp.bfloat16)


def _last_body(h_ref, mc_ref, m_ref, lfs_ref, lfb_ref, *refs):
    out_ref = refs[-1]
    h = h_ref[0].astype(jnp.float32)
    mc = mc_ref[0]
    m = m_ref[0]
    h3 = _attn_ffn_core(h, mc, *refs[:-1])
    hf = _ln_tc(h3, lfs_ref[...], lfb_ref[...])
    pooled = jnp.dot(m, hf, preferred_element_type=jnp.float32)
    denom = jnp.maximum(jnp.sum(m), 1.0)
    out_ref[0] = pooled * (1.0 / denom)


def _full_spec(shp):
    return pl.BlockSpec(shp, lambda b: (0,) * len(shp))


def _wspecs():
    return [
        _full_spec((D, D)), _full_spec((D, D)), _full_spec((D, D)),
        _full_spec((D, D)),
        _full_spec((D,)), _full_spec((D,)), _full_spec((D,)),
        _full_spec((D,)),
        _full_spec((D, DFF)), _full_spec((DFF,)), _full_spec((DFF, D)),
        _full_spec((D,)),
    ]


def _wargs(wq, wk, wv, wo, l1s, l1b, l2s, l2b, w1, b1, w2, b2):
    bf = jnp.bfloat16
    return (wq.astype(bf), wk.astype(bf), wv.astype(bf), wo.astype(bf),
            l1s, l1b, l2s, l2b, w1.astype(bf), b1, w2.astype(bf), b2)


def _first_layer(table, mcol, w_pad, b_proj, *w):
    B = table.shape[0]
    return pl.pallas_call(
        _first_body,
        grid=(B,),
        compiler_params=pltpu.CompilerParams(
            dimension_semantics=("parallel",)),
        in_specs=[
            pl.BlockSpec((1, MAX_EVENTS, TABW), lambda b: (b, 0, 0)),
            pl.BlockSpec((1, MAX_EVENTS, 1), lambda b: (b, 0, 0)),
            _full_spec((TABW, D)), _full_spec((D,)),
        ] + _wspecs(),
        out_specs=pl.BlockSpec((1, MAX_EVENTS, D), lambda b: (b, 0, 0)),
        out_shape=jax.ShapeDtypeStruct((B, MAX_EVENTS, D), jnp.bfloat16),
    )(table, mcol, w_pad, b_proj, *_wargs(*w))


def _mid_layer(h, mcol, *w):
    B = h.shape[0]
    return pl.pallas_call(
        _mid_body,
        grid=(B,),
        compiler_params=pltpu.CompilerParams(
            dimension_semantics=("parallel",)),
        in_specs=[
            pl.BlockSpec((1, MAX_EVENTS, D), lambda b: (b, 0, 0)),
            pl.BlockSpec((1, MAX_EVENTS, 1), lambda b: (b, 0, 0)),
        ] + _wspecs(),
        out_specs=pl.BlockSpec((1, MAX_EVENTS, D), lambda b: (b, 0, 0)),
        out_shape=jax.ShapeDtypeStruct((B, MAX_EVENTS, D), jnp.bfloat16),
    )(h, mcol, *_wargs(*w))


def _last_layer(h, mcol, m3, lnf_s, lnf_b, *w):
    B = h.shape[0]
    return pl.pallas_call(
        _last_body,
        grid=(B,),
        compiler_params=pltpu.CompilerParams(
            dimension_semantics=("parallel",)),
        in_specs=[
            pl.BlockSpec((1, MAX_EVENTS, D), lambda b: (b, 0, 0)),
            pl.BlockSpec((1, MAX_EVENTS, 1), lambda b: (b, 0, 0)),
            pl.BlockSpec((1, 1, MAX_EVENTS), lambda b: (b, 0, 0)),
            _full_spec((D,)), _full_spec((D,)),
        ] + _wspecs(),
        out_specs=pl.BlockSpec((1, 1, D), lambda b: (b, 0, 0)),
        out_shape=jax.ShapeDtypeStruct((B, 1, D), jnp.float32),
    )(h, mcol, m3, lnf_s, lnf_b, *_wargs(*w))


def kernel(x, W_proj, b_proj, Wq, Wk, Wv, Wo, ln1_s, ln1_b, ln2_s, ln2_b,
           W1, b1, W2, b2, lnf_s, lnf_b):
    B = x.shape[0]
    xf = x.astype(jnp.float32).reshape(B, NTIME * NCOMP)
    ztab = jnp.zeros((MAX_EVENTS * TABW,), jnp.float32)
    zmask = jnp.zeros((MAX_EVENTS,), jnp.float32)
    tabflat, mask = _sc_event_build(xf, ztab, zmask)
    table = tabflat.reshape(B, MAX_EVENTS, TABW)
    m3 = mask.reshape(B, 1, MAX_EVENTS)
    mcol = mask.reshape(B, MAX_EVENTS, 1)

    w_pad = jnp.zeros((TABW, D), jnp.float32).at[:W_proj.shape[0]].set(W_proj)
    wl = lambda l: (Wq[l], Wk[l], Wv[l], Wo[l], ln1_s[l], ln1_b[l],
                    ln2_s[l], ln2_b[l], W1[l], b1[l], W2[l], b2[l])
    h = _first_layer(table, mcol, w_pad, b_proj, *wl(0))
    for l in range(1, L - 1):
        h = _mid_layer(h, mcol, *wl(l))
    out = _last_layer(h, mcol, m3, lnf_s, lnf_b, *wl(L - 1))
    return out.reshape(B, D)


# speedup vs baseline: 1.0014x; 1.0014x over previous
import functools

import jax
import jax.numpy as jnp
from jax import lax
from jax.experimental import pallas as pl
from jax.experimental.pallas import tpu as pltpu
from jax.experimental.pallas import tpu_sc as plsc

NTIME = 512
NCOMP = 16
MAX_EVENTS = 1024
D = 512
L = 4
H = 8
DH = D // H
DFF = 2048
TABW = 32


def _sc_event_build(xf, ztab, zmask):
    B = xf.shape[0]
    mesh = plsc.VectorSubcoreMesh(core_axis_name="c", subcore_axis_name="s")

    @functools.partial(
        pl.kernel,
        mesh=mesh,
        compiler_params=pltpu.CompilerParams(needs_layout_passes=False),
        out_type=(
            jax.ShapeDtypeStruct((B, MAX_EVENTS * TABW), jnp.float32),
            jax.ShapeDtypeStruct((B, MAX_EVENTS), jnp.float32),
        ),
        scratch_types=[
            pltpu.VMEM((NTIME * NCOMP,), jnp.float32),
            pltpu.VMEM((MAX_EVENTS * TABW,), jnp.float32),
            pltpu.VMEM((MAX_EVENTS,), jnp.float32),
        ],
    )
    def k(x_hbm, ztab_hbm, zmask_hbm, tab_hbm, mask_hbm, x_v, tab_v, msk_v):
        wid = lax.axis_index("s") * 2 + lax.axis_index("c")

        @pl.when(wid < B)
        def _():
            pltpu.sync_copy(x_hbm.at[wid], x_v)
            pltpu.sync_copy(ztab_hbm, tab_v)
            pltpu.sync_copy(zmask_hbm, msk_v)

            lanes = lax.iota(jnp.int32, 16)
            ones16 = jnp.ones((16,), jnp.float32)

            def body(t, carry):
                base, prev, last_start, last_rank = carry
                xv = x_v[pl.ds(t * 16, 16)]
                bits = (xv > 0.5).astype(jnp.int32)
                st = bits != prev
                sti = st.astype(jnp.int32)
                inc = plsc.cumsum(sti)
                rank = base + inc - sti
                valid = jnp.logical_and(st, rank < MAX_EVENTS)
                pm = jnp.logical_and(
                    st,
                    jnp.logical_and(last_rank >= 0, last_rank < MAX_EVENTS))
                durv = (t - last_start).astype(jnp.float32) * (1.0 / NTIME)
                plsc.store_scatter(tab_v, [last_rank * TABW + 18], durv,
                                   mask=pm)
                fl = rank * TABW
                plsc.store_scatter(tab_v, [fl + lanes], ones16, mask=valid)
                plsc.store_scatter(tab_v, [fl + 16],
                                   bits.astype(jnp.float32), mask=valid)
                tn = jnp.full((16,), 0.0, jnp.float32) + (
                    t.astype(jnp.float32) * (1.0 / (NTIME - 1)))
                plsc.store_scatter(tab_v, [fl + 17], tn, mask=valid)
                plsc.store_scatter(msk_v, [rank], ones16, mask=valid)
                nbase = base + jnp.sum(sti)
                nstart = jnp.where(st, t, last_start)
                nrank = jnp.where(st, rank, last_rank)
                return (nbase, bits, nstart, nrank)

            init = (jnp.int32(0),
                    jnp.full((16,), -1, jnp.int32),
                    jnp.zeros((16,), jnp.int32),
                    jnp.full((16,), -1, jnp.int32))
            base, prev, last_start, last_rank = lax.fori_loop(
                0, NTIME, body, init)
            pm = jnp.logical_and(last_rank >= 0, last_rank < MAX_EVENTS)
            durv = (NTIME - last_start).astype(jnp.float32) * (1.0 / NTIME)
            plsc.store_scatter(tab_v, [last_rank * TABW + 18], durv, mask=pm)

            pltpu.sync_copy(tab_v, tab_hbm.at[wid])
            pltpu.sync_copy(msk_v, mask_hbm.at[wid])

    return k(xf, ztab, zmask)


def _ln_tc(x, s, b):
    mu = jnp.mean(x, axis=-1, keepdims=True)
    var = jnp.mean((x - mu) ** 2, axis=-1, keepdims=True)
    return (x - mu) * lax.rsqrt(var + 1e-5) * s[None, :] + b[None, :]


def _attn_ffn_core(h, mc, wq_ref, wk_ref, wv_ref, wo_ref,
                   l1s_ref, l1b_ref, l2s_ref, l2b_ref,
                   w1_ref, b1_ref, w2_ref, b2_ref):
    bf = jnp.bfloat16
    hn = _ln_tc(h, l1s_ref[...], l1b_ref[...]).astype(bf)
    q = jnp.dot(hn, wq_ref[...], preferred_element_type=jnp.float32)
    k = jnp.dot(hn, wk_ref[...], preferred_element_type=jnp.float32)
    v = jnp.dot(hn, wv_ref[...], preferred_element_type=jnp.float32)
    vm = v * mc
    scale = 1.0 / (DH ** 0.5)
    qb = (q * scale).astype(bf)
    kb = k.astype(bf)
    outs = []
    for hh in range(H):
        sl = slice(hh * DH, (hh + 1) * DH)
        s = lax.dot_general(qb[:, sl], kb[:, sl],
                            (((1,), (1,)), ((), ())),
                            preferred_element_type=jnp.float32)
        eb = jnp.exp(s).astype(bf)
        va = jnp.concatenate([vm[:, sl], mc], axis=1).astype(bf)
        oa = jnp.dot(eb, va, preferred_element_type=jnp.float32)
        outs.append(oa[:, :DH] * (1.0 / oa[:, DH:DH + 1]))
    o = jnp.concatenate(outs, axis=1).astype(bf)
    h2 = h + jnp.dot(o, wo_ref[...], preferred_element_type=jnp.float32)
    hn2 = _ln_tc(h2, l2s_ref[...], l2b_ref[...]).astype(bf)
    t1 = jnp.dot(hn2, w1_ref[...], preferred_element_type=jnp.float32)
    t1 = jnp.maximum(t1 + b1_ref[...][None, :], 0.0).astype(bf)
    t2 = jnp.dot(t1, w2_ref[...], preferred_element_type=jnp.float32)
    return h2 + t2 + b2_ref[...][None, :]


def _first_body(tab_ref, mc_ref, wp_ref, bp_ref, *refs):
    out_ref = refs[-1]
    tab = tab_ref[0]
    mc = mc_ref[0]
    h = jnp.dot(tab, wp_ref[...], preferred_element_type=jnp.float32)
    h = h + bp_ref[...][None, :]
    out_ref[0] = _attn_ffn_core(h, mc, *refs[:-1]).astype(jnp.bfloat16)


def _mid_body(h_ref, mc_ref, *refs):
    out_ref = refs[-1]
    h = h_ref[0].astype(jnp.float32)
    mc = mc_ref[0]
    out_ref[0] = _attn_ffn_core(h, mc, *refs[:-1]).astype(jnp.bfloat16)


def _last_body(h_ref, mc_ref, m_ref, lfs_ref, lfb_ref, *refs):
    out_ref = refs[-1]
    h = h_ref[0].astype(jnp.float32)
    mc = mc_ref[0]
    m = m_ref[0]
    h3 = _attn_ffn_core(h, mc, *refs[:-1])
    hf = _ln_tc(h3, lfs_ref[...], lfb_ref[...])
    pooled = jnp.dot(m, hf, preferred_element_type=jnp.float32)
    denom = jnp.maximum(jnp.sum(m), 1.0)
    out_ref[0] = pooled * (1.0 / denom)


def _full_spec(shp):
    return pl.BlockSpec(shp, lambda b: (0,) * len(shp))


def _wspecs():
    return [
        _full_spec((D, D)), _full_spec((D, D)), _full_spec((D, D)),
        _full_spec((D, D)),
        _full_spec((D,)), _full_spec((D,)), _full_spec((D,)),
        _full_spec((D,)),
        _full_spec((D, DFF)), _full_spec((DFF,)), _full_spec((DFF, D)),
        _full_spec((D,)),
    ]


def _wargs(wq, wk, wv, wo, l1s, l1b, l2s, l2b, w1, b1, w2, b2):
    bf = jnp.bfloat16
    return (wq.astype(bf), wk.astype(bf), wv.astype(bf), wo.astype(bf),
            l1s, l1b, l2s, l2b, w1.astype(bf), b1, w2.astype(bf), b2)


def _first_layer(table, mcol, w_pad, b_proj, *w):
    B = table.shape[0]
    return pl.pallas_call(
        _first_body,
        grid=(B,),
        in_specs=[
            pl.BlockSpec((1, MAX_EVENTS, TABW), lambda b: (b, 0, 0)),
            pl.BlockSpec((1, MAX_EVENTS, 1), lambda b: (b, 0, 0)),
            _full_spec((TABW, D)), _full_spec((D,)),
        ] + _wspecs(),
        out_specs=pl.BlockSpec((1, MAX_EVENTS, D), lambda b: (b, 0, 0)),
        out_shape=jax.ShapeDtypeStruct((B, MAX_EVENTS, D), jnp.bfloat16),
    )(table, mcol, w_pad, b_proj, *_wargs(*w))


def _mid_layer(h, mcol, *w):
    B = h.shape[0]
    return pl.pallas_call(
        _mid_body,
        grid=(B,),
        in_specs=[
            pl.BlockSpec((1, MAX_EVENTS, D), lambda b: (b, 0, 0)),
            pl.BlockSpec((1, MAX_EVENTS, 1), lambda b: (b, 0, 0)),
        ] + _wspecs(),
        out_specs=pl.BlockSpec((1, MAX_EVENTS, D), lambda b: (b, 0, 0)),
        out_shape=jax.ShapeDtypeStruct((B, MAX_EVENTS, D), jnp.bfloat16),
    )(h, mcol, *_wargs(*w))


def _last_layer(h, mcol, m3, lnf_s, lnf_b, *w):
    B = h.shape[0]
    return pl.pallas_call(
        _last_body,
        grid=(B,),
        in_specs=[
            pl.BlockSpec((1, MAX_EVENTS, D), lambda b: (b, 0, 0)),
            pl.BlockSpec((1, MAX_EVENTS, 1), lambda b: (b, 0, 0)),
            pl.BlockSpec((1, 1, MAX_EVENTS), lambda b: (b, 0, 0)),
            _full_spec((D,)), _full_spec((D,)),
        ] + _wspecs(),
        out_specs=pl.BlockSpec((1, 1, D), lambda b: (b, 0, 0)),
        out_shape=jax.ShapeDtypeStruct((B, 1, D), jnp.float32),
    )(h, mcol, m3, lnf_s, lnf_b, *_wargs(*w))


def kernel(x, W_proj, b_proj, Wq, Wk, Wv, Wo, ln1_s, ln1_b, ln2_s, ln2_b,
           W1, b1, W2, b2, lnf_s, lnf_b):
    B = x.shape[0]
    xf = x.astype(jnp.float32).reshape(B, NTIME * NCOMP)
    ztab = jnp.zeros((MAX_EVENTS * TABW,), jnp.float32)
    zmask = jnp.zeros((MAX_EVENTS,), jnp.float32)
    tabflat, mask = _sc_event_build(xf, ztab, zmask)
    table = tabflat.reshape(B, MAX_EVENTS, TABW)
    m3 = mask.reshape(B, 1, MAX_EVENTS)
    mcol = mask.reshape(B, MAX_EVENTS, 1)

    w_pad = jnp.zeros((TABW, D), jnp.float32).at[:W_proj.shape[0]].set(W_proj)
    wl = lambda l: (Wq[l], Wk[l], Wv[l], Wo[l], ln1_s[l], ln1_b[l],
                    ln2_s[l], ln2_b[l], W1[l], b1[l], W2[l], b2[l])
    h = _first_layer(table, mcol, w_pad, b_proj, *wl(0))
    for l in range(1, L - 1):
        h = _mid_layer(h, mcol, *wl(l))
    out = _last_layer(h, mcol, m3, lnf_s, lnf_b, *wl(L - 1))
    return out.reshape(B, D)
